# baseline (speedup 1.0000x reference)
import functools

import jax
import jax.numpy as jnp
from jax import lax
from jax.experimental import pallas as pl
from jax.experimental.pallas import tpu as pltpu
from jax.experimental.pallas import tpu_sc as plsc

_NC = 2
_NS = 16
_NW = _NC * _NS
_CH = 128
_G = 16

_mesh = plsc.VectorSubcoreMesh(core_axis_name="c", subcore_axis_name="s")
_SPLIT0 = 0.2


def _sc_degree(dst2d, zeros16, ones16, *, npad, k, w):
    zrows = npad // _NS

    @functools.partial(
        pl.kernel,
        out_type=jax.ShapeDtypeStruct((_NC, npad, w), jnp.float32),
        mesh=_mesh,
        scratch_types=[
            pltpu.VMEM((k, _CH), jnp.int32),
            pltpu.VMEM((_CH, w), jnp.float32),
            pltpu.VMEM_SHARED((npad, w), jnp.float32),
        ],
    )
    def kern(dst_hbm, z_hbm, ones_hbm, out_hbm, idx_v, ones_v, acc_sh):
        c = lax.axis_index("c")
        s = lax.axis_index("s")
        wid = s * _NC + c
        pltpu.sync_copy(dst_hbm.at[pl.ds(wid * k, k)], idx_v)
        pltpu.sync_copy(ones_hbm, ones_v)
        pltpu.sync_copy(z_hbm.at[pl.ds(s * zrows, zrows)],
                        acc_sh.at[pl.ds(s * zrows, zrows)])
        plsc.subcore_barrier()

        @pl.loop(0, k)
        def _(j):
            pltpu.sync_copy(ones_v, acc_sh.at[idx_v.at[j]], add=True)

        plsc.subcore_barrier()
        pltpu.sync_copy(acc_sh.at[pl.ds(s * zrows, zrows)],
                        out_hbm.at[c, pl.ds(s * zrows, zrows)])

    return kern(dst2d, zeros16, ones16)


def _sc_msg(g, src2d, dst2d, zerosd, *, npad, k0, k1, d):
    zrows = npad // _NS

    @functools.partial(
        pl.kernel,
        out_type=jax.ShapeDtypeStruct((_NC, npad, d), jnp.float32),
        mesh=_mesh,
        scratch_types=[
            pltpu.VMEM((_G, _CH), jnp.int32),
            pltpu.VMEM((_G, _CH), jnp.int32),
            pltpu.VMEM((_CH, d), jnp.float32),
            pltpu.VMEM((_CH, d), jnp.float32),
            pltpu.VMEM_SHARED((npad, d), jnp.float32),
            pltpu.SemaphoreType.DMA,
            pltpu.SemaphoreType.DMA,
        ],
    )
    def kern(g_hbm, src_hbm, dst_hbm, z_hbm, out_hbm,
             si_v, di_v, rows0, rows1, acc_sh, sem0, sem1):
        c = lax.axis_index("c")
        s = lax.axis_index("s")
        kc = jnp.where(c == 0, k0, k1)
        row0 = jnp.where(c == 0, s * k0, _NS * k0 + s * k1)
        bufs = (rows0, rows1)
        sems = (sem0, sem1)
        pltpu.sync_copy(z_hbm.at[pl.ds(s * zrows, zrows)],
                        acc_sh.at[pl.ds(s * zrows, zrows)])
        plsc.subcore_barrier()

        @pl.loop(0, kc // _G)
        def _(grp):
            base = pl.multiple_of(row0 + grp * _G, 8)
            pltpu.sync_copy(src_hbm.at[pl.ds(base, _G)], si_v)
            pltpu.sync_copy(dst_hbm.at[pl.ds(base, _G)], di_v)

            pltpu.async_copy(g_hbm.at[si_v.at[0]], bufs[0], sems[0])
            pltpu.async_copy(g_hbm.at[si_v.at[1]], bufs[1], sems[1])

            @pl.loop(0, _G, step=2)
            def _(j0):
                for b in range(2):
                    j = j0 + b
                    nxt = jnp.minimum(j + 2, _G - 1)
                    pltpu.make_async_copy(
                        g_hbm.at[si_v.at[j]], bufs[b], sems[b]).wait()
                    pltpu.sync_copy(bufs[b], acc_sh.at[di_v.at[j]], add=True)
                    pltpu.async_copy(g_hbm.at[si_v.at[nxt]], bufs[b], sems[b])

            for b in range(2):
                pltpu.make_async_copy(
                    g_hbm.at[si_v.at[_G - 1]], bufs[b], sems[b]).wait()

        plsc.subcore_barrier()
        pltpu.sync_copy(acc_sh.at[pl.ds(s * zrows, zrows)],
                        out_hbm.at[c, pl.ds(s * zrows, zrows)])

    return kern(g, src2d, dst2d, zerosd)


def _tc_first(nodes, w1, degp, *, n, d):
    def body(x_ref, w_ref, degp_ref, g_ref):
        deg = degp_ref[0, :n, :1] + degp_ref[1, :n, :1] + 1.0
        dis = lax.rsqrt(deg)
        h = jnp.dot(x_ref[...], w_ref[...], preferred_element_type=jnp.float32)
        g_ref[...] = h * dis

    return pl.pallas_call(
        body,
        out_shape=jax.ShapeDtypeStruct((n, d), jnp.float32),
    )(nodes, w1, degp)


def _tc_mid(acc, g1, b1, degp, w2, *, n, d):
    def body(acc_ref, g_ref, b_ref, degp_ref, w_ref, g2_ref):
        deg = degp_ref[0, :n, :1] + degp_ref[1, :n, :1] + 1.0
        dis = lax.rsqrt(deg)
        tot = (acc_ref[0, :n] + acc_ref[1, :n] + g_ref[...]) * dis + b_ref[...]
        h = jnp.maximum(tot, 0.0)
        g2_ref[...] = jnp.dot(h, w_ref[...],
                              preferred_element_type=jnp.float32) * dis

    return pl.pallas_call(
        body,
        out_shape=jax.ShapeDtypeStruct((n, d), jnp.float32),
    )(acc, g1, b1, degp, w2)


def _tc_last(acc, g2, b2, degp, *, n, d):
    def body(acc_ref, g_ref, b_ref, degp_ref, out_ref):
        deg = degp_ref[0, :n, :1] + degp_ref[1, :n, :1] + 1.0
        dis = lax.rsqrt(deg)
        tot = (acc_ref[0, :n] + acc_ref[1, :n] + g_ref[...]) * dis + b_ref[...]
        out_ref[...] = jnp.maximum(tot, 0.0)

    return pl.pallas_call(
        body,
        out_shape=jax.ShapeDtypeStruct((n, d), jnp.float32),
    )(acc, g2, b2, degp)


def kernel(graphs, x, W1, b1, W2, b2):
    seq_len, bsz, d = x.shape
    n = seq_len * bsz
    e = graphs.shape[0] * graphs.shape[2]

    nodes = jnp.transpose(x, (1, 0, 2)).reshape(-1, d)
    offs = (jnp.arange(bsz, dtype=graphs.dtype) * seq_len)[:, None, None]
    ei = jnp.transpose(graphs + offs, (1, 0, 2)).reshape(2, -1)
    src = ei[0].astype(jnp.int32)
    dst = ei[1].astype(jnp.int32)

    ksum = -(-e // (_NS * _CH))
    ksum = -(-ksum // (2 * _G)) * (2 * _G)
    k0 = max(_G, int(round(ksum * _SPLIT0 / _G)) * _G)
    k1 = ksum - k0
    ep = _NS * ksum * _CH
    kdeg = ksum // 2
    npad = -(-(n + 1) // (_NS * 8)) * (_NS * 8)
    src2d = jnp.concatenate(
        [src, jnp.zeros((ep - e,), jnp.int32)]).reshape(_NS * ksum, _CH)
    dst2d = jnp.concatenate(
        [dst, jnp.full((ep - e,), n, jnp.int32)]).reshape(_NS * ksum, _CH)

    zeros16 = jnp.zeros((npad, d), jnp.float32)
    ones16 = jnp.ones((_CH, d), jnp.float32)
    zerosd = jnp.zeros((npad, d), jnp.float32)

    degp = _sc_degree(dst2d, zeros16, ones16, npad=npad, k=kdeg, w=d)
    g1 = _tc_first(nodes, W1, degp, n=n, d=d)
    acc1 = _sc_msg(g1, src2d, dst2d, zerosd, npad=npad, k0=k0, k1=k1, d=d)
    g2 = _tc_mid(acc1, g1, b1, degp, W2, n=n, d=d)
    acc2 = _sc_msg(g2, src2d, dst2d, zerosd, npad=npad, k0=k0, k1=k1, d=d)
    out = _tc_last(acc2, g2, b2, degp, n=n, d=d)
    return jnp.transpose(out.reshape(bsz, seq_len, d), (1, 0, 2))


# speedup vs baseline: 1.1361x; 1.1361x over previous
import functools

import jax
import jax.numpy as jnp
from jax import lax
from jax.experimental import pallas as pl
from jax.experimental.pallas import tpu as pltpu
from jax.experimental.pallas import tpu_sc as plsc

_NC = 2
_NS = 16
_NW = _NC * _NS
_CH = 128
_G = 16

_mesh = plsc.VectorSubcoreMesh(core_axis_name="c", subcore_axis_name="s")
_SPLIT0 = 0.8


def _sc_degree(dst2d, zeros16, ones16, *, npad, k, w):
    zrows = npad // _NS

    @functools.partial(
        pl.kernel,
        out_type=jax.ShapeDtypeStruct((_NC, npad, w), jnp.float32),
        mesh=_mesh,
        scratch_types=[
            pltpu.VMEM((k, _CH), jnp.int32),
            pltpu.VMEM((_CH, w), jnp.float32),
            pltpu.VMEM_SHARED((npad, w), jnp.float32),
        ],
    )
    def kern(dst_hbm, z_hbm, ones_hbm, out_hbm, idx_v, ones_v, acc_sh):
        c = lax.axis_index("c")
        s = lax.axis_index("s")
        wid = s * _NC + c
        pltpu.sync_copy(dst_hbm.at[pl.ds(wid * k, k)], idx_v)
        pltpu.sync_copy(ones_hbm, ones_v)
        pltpu.sync_copy(z_hbm.at[pl.ds(s * zrows, zrows)],
                        acc_sh.at[pl.ds(s * zrows, zrows)])
        plsc.subcore_barrier()

        @pl.loop(0, k)
        def _(j):
            pltpu.sync_copy(ones_v, acc_sh.at[idx_v.at[j]], add=True)

        plsc.subcore_barrier()
        pltpu.sync_copy(acc_sh.at[pl.ds(s * zrows, zrows)],
                        out_hbm.at[c, pl.ds(s * zrows, zrows)])

    return kern(dst2d, zeros16, ones16)


def _sc_msg(g, src2d, dst2d, zerosd, *, npad, k0, k1, d):
    zrows = npad // _NS

    @functools.partial(
        pl.kernel,
        out_type=jax.ShapeDtypeStruct((_NC, npad, d), jnp.float32),
        mesh=_mesh,
        scratch_types=[
            pltpu.VMEM((_G, _CH), jnp.int32),
            pltpu.VMEM((_G, _CH), jnp.int32),
            pltpu.VMEM((_CH, d), jnp.float32),
            pltpu.VMEM((_CH, d), jnp.float32),
            pltpu.VMEM_SHARED((npad, d), jnp.float32),
            pltpu.SemaphoreType.DMA,
            pltpu.SemaphoreType.DMA,
        ],
    )
    def kern(g_hbm, src_hbm, dst_hbm, z_hbm, out_hbm,
             si_v, di_v, rows0, rows1, acc_sh, sem0, sem1):
        c = lax.axis_index("c")
        s = lax.axis_index("s")
        kc = jnp.where(c == 0, k0, k1)
        row0 = jnp.where(c == 0, s * k0, _NS * k0 + s * k1)
        bufs = (rows0, rows1)
        sems = (sem0, sem1)
        pltpu.sync_copy(z_hbm.at[pl.ds(s * zrows, zrows)],
                        acc_sh.at[pl.ds(s * zrows, zrows)])
        plsc.subcore_barrier()

        @pl.loop(0, kc // _G)
        def _(grp):
            base = pl.multiple_of(row0 + grp * _G, 8)
            pltpu.sync_copy(src_hbm.at[pl.ds(base, _G)], si_v)
            pltpu.sync_copy(dst_hbm.at[pl.ds(base, _G)], di_v)

            pltpu.async_copy(g_hbm.at[si_v.at[0]], bufs[0], sems[0])
            pltpu.async_copy(g_hbm.at[si_v.at[1]], bufs[1], sems[1])

            @pl.loop(0, _G, step=2)
            def _(j0):
                for b in range(2):
                    j = j0 + b
                    nxt = jnp.minimum(j + 2, _G - 1)
                    pltpu.make_async_copy(
                        g_hbm.at[si_v.at[j]], bufs[b], sems[b]).wait()
                    pltpu.sync_copy(bufs[b], acc_sh.at[di_v.at[j]], add=True)
                    pltpu.async_copy(g_hbm.at[si_v.at[nxt]], bufs[b], sems[b])

            for b in range(2):
                pltpu.make_async_copy(
                    g_hbm.at[si_v.at[_G - 1]], bufs[b], sems[b]).wait()

        plsc.subcore_barrier()
        pltpu.sync_copy(acc_sh.at[pl.ds(s * zrows, zrows)],
                        out_hbm.at[c, pl.ds(s * zrows, zrows)])

    return kern(g, src2d, dst2d, zerosd)


def _tc_first(nodes, w1, degp, *, n, d):
    def body(x_ref, w_ref, degp_ref, g_ref):
        deg = degp_ref[0, :n, :1] + degp_ref[1, :n, :1] + 1.0
        dis = lax.rsqrt(deg)
        h = jnp.dot(x_ref[...], w_ref[...], preferred_element_type=jnp.float32)
        g_ref[...] = h * dis

    return pl.pallas_call(
        body,
        out_shape=jax.ShapeDtypeStruct((n, d), jnp.float32),
    )(nodes, w1, degp)


def _tc_mid(acc, g1, b1, degp, w2, *, n, d):
    def body(acc_ref, g_ref, b_ref, degp_ref, w_ref, g2_ref):
        deg = degp_ref[0, :n, :1] + degp_ref[1, :n, :1] + 1.0
        dis = lax.rsqrt(deg)
        tot = (acc_ref[0, :n] + acc_ref[1, :n] + g_ref[...]) * dis + b_ref[...]
        h = jnp.maximum(tot, 0.0)
        g2_ref[...] = jnp.dot(h, w_ref[...],
                              preferred_element_type=jnp.float32) * dis

    return pl.pallas_call(
        body,
        out_shape=jax.ShapeDtypeStruct((n, d), jnp.float32),
    )(acc, g1, b1, degp, w2)


def _tc_last(acc, g2, b2, degp, *, n, d):
    def body(acc_ref, g_ref, b_ref, degp_ref, out_ref):
        deg = degp_ref[0, :n, :1] + degp_ref[1, :n, :1] + 1.0
        dis = lax.rsqrt(deg)
        tot = (acc_ref[0, :n] + acc_ref[1, :n] + g_ref[...]) * dis + b_ref[...]
        out_ref[...] = jnp.maximum(tot, 0.0)

    return pl.pallas_call(
        body,
        out_shape=jax.ShapeDtypeStruct((n, d), jnp.float32),
    )(acc, g2, b2, degp)


def kernel(graphs, x, W1, b1, W2, b2):
    seq_len, bsz, d = x.shape
    n = seq_len * bsz
    e = graphs.shape[0] * graphs.shape[2]

    nodes = jnp.transpose(x, (1, 0, 2)).reshape(-1, d)
    offs = (jnp.arange(bsz, dtype=graphs.dtype) * seq_len)[:, None, None]
    ei = jnp.transpose(graphs + offs, (1, 0, 2)).reshape(2, -1)
    src = ei[0].astype(jnp.int32)
    dst = ei[1].astype(jnp.int32)

    ksum = -(-e // (_NS * _CH))
    ksum = -(-ksum // (2 * _G)) * (2 * _G)
    k0 = max(_G, int(round(ksum * _SPLIT0 / _G)) * _G)
    k1 = ksum - k0
    ep = _NS * ksum * _CH
    kdeg = ksum // 2
    npad = -(-(n + 1) // (_NS * 8)) * (_NS * 8)
    src2d = jnp.concatenate(
        [src, jnp.zeros((ep - e,), jnp.int32)]).reshape(_NS * ksum, _CH)
    dst2d = jnp.concatenate(
        [dst, jnp.full((ep - e,), n, jnp.int32)]).reshape(_NS * ksum, _CH)

    zeros16 = jnp.zeros((npad, d), jnp.float32)
    ones16 = jnp.ones((_CH, d), jnp.float32)
    zerosd = jnp.zeros((npad, d), jnp.float32)

    degp = _sc_degree(dst2d, zeros16, ones16, npad=npad, k=kdeg, w=d)
    g1 = _tc_first(nodes, W1, degp, n=n, d=d)
    acc1 = _sc_msg(g1, src2d, dst2d, zerosd, npad=npad, k0=k0, k1=k1, d=d)
    g2 = _tc_mid(acc1, g1, b1, degp, W2, n=n, d=d)
    acc2 = _sc_msg(g2, src2d, dst2d, zerosd, npad=npad, k0=k0, k1=k1, d=d)
    out = _tc_last(acc2, g2, b2, degp, n=n, d=d)
    return jnp.transpose(out.reshape(bsz, seq_len, d), (1, 0, 2))
